# baseline (speedup 1.0000x reference)
import functools

import jax
import jax.numpy as jnp
from jax import lax
from jax.experimental import pallas as pl
from jax.experimental.pallas import tpu as pltpu
from jax.experimental.pallas import tpu_sc as plsc

B, S, H, V = 4, 2048, 1024, 50257
NC, NS = 2, 16
NW = NC * NS
SEQ_PER_W = S // NW
P = 8
NCH = SEQ_PER_W // P
LANES = 16
UNROLL = 16


def _emb_body(x_hbm, tab_hbm, pos_hbm, out_hbm, idx_v, sb_v, pos_v, sem0, sem1):
    wid = lax.axis_index("s") * NC + lax.axis_index("c")
    s0 = wid * SEQ_PER_W
    sems = (sem0, sem1)

    for b in range(B):
        pltpu.sync_copy(x_hbm.at[pl.ds(b * S + s0, SEQ_PER_W)],
                        idx_v.at[pl.ds(b * SEQ_PER_W, SEQ_PER_W)])

    def descs(c, buf):
        d = [pltpu.make_async_copy(pos_hbm.at[pl.ds(s0 + c * P, P)],
                                   pos_v.at[buf], sems[buf])]
        for b in range(B):
            d.append(pltpu.make_async_copy(
                tab_hbm.at[idx_v.at[pl.ds(b * SEQ_PER_W + c * P, P)]],
                sb_v.at[buf, b], sems[buf]))
        return d

    def start(c, buf):
        for d in descs(c, buf):
            d.start()

    start(0, 0)
    start(1, 1)

    def pair_body(i, _):
        for sub in range(2):
            c = 2 * i + sub
            buf = sub
            for d in descs(c, buf):
                d.wait()

            def row_body(r, _):
                def col_body(jj, _):
                    for u in range(UNROLL):
                        sl = pl.ds((jj * UNROLL + u) * LANES, LANES)
                        p = pos_v[buf, r, sl]
                        for b in range(B):
                            sb_v[buf, b, r, sl] = sb_v[buf, b, r, sl] + p
                    return 0

                lax.fori_loop(0, H // (LANES * UNROLL), col_body, 0)
                return 0

            lax.fori_loop(0, P, row_body, 0)

            for b in range(B):
                pltpu.sync_copy(sb_v.at[buf, b],
                                out_hbm.at[pl.ds(b * S + s0 + c * P, P)])

            @pl.when(c + 2 < NCH)
            def _():
                start(c + 2, buf)
        return 0

    lax.fori_loop(0, NCH // 2, pair_body, 0)


@jax.jit
def _emb(x_flat, table, pos):
    mesh = plsc.VectorSubcoreMesh(core_axis_name="c", subcore_axis_name="s")
    f = functools.partial(
        pl.kernel,
        mesh=mesh,
        out_type=jax.ShapeDtypeStruct((B * S, H), jnp.float32),
        scratch_types=[
            pltpu.VMEM((B * SEQ_PER_W,), jnp.int32),
            pltpu.VMEM((2, B, P, H), jnp.float32),
            pltpu.VMEM((2, P, H), jnp.float32),
            pltpu.SemaphoreType.DMA,
            pltpu.SemaphoreType.DMA,
        ],
    )(_emb_body)
    return f(x_flat, table, pos)


def kernel(x, token_table, pos_emb):
    x_flat = x.reshape(-1).astype(jnp.int32)
    pos = pos_emb.reshape(S, H)
    out = _emb(x_flat, token_table, pos)
    return out.reshape(B, S, H)


# speedup vs baseline: 1.6833x; 1.6833x over previous
import functools

import jax
import jax.numpy as jnp
from jax import lax
from jax.experimental import pallas as pl
from jax.experimental.pallas import tpu as pltpu
from jax.experimental.pallas import tpu_sc as plsc

B, S, H, V = 4, 2048, 1024, 50257
NC, NS = 2, 16
NW = NC * NS
SEQ_PER_W = S // NW
P = 8
NCH = SEQ_PER_W // P
LANES = 16


def _emb_body(x_hbm, tab_hbm, pos_hbm, out_hbm, idx_v, sb_v, pos_v, sem0, sem1):
    wid = lax.axis_index("s") * NC + lax.axis_index("c")
    s0 = wid * SEQ_PER_W
    sems = (sem0, sem1)

    for b in range(B):
        pltpu.sync_copy(x_hbm.at[b, pl.ds(s0, SEQ_PER_W)],
                        idx_v.at[pl.ds(b * SEQ_PER_W, SEQ_PER_W)])

    def descs(c, buf):
        d = [pltpu.make_async_copy(pos_hbm.at[pl.ds(s0 + c * P, P)],
                                   pos_v.at[buf], sems[buf])]
        for b in range(B):
            d.append(pltpu.make_async_copy(
                tab_hbm.at[idx_v.at[pl.ds(b * SEQ_PER_W + c * P, P)]],
                sb_v.at[buf, b], sems[buf]))
        return d

    def start(c, buf):
        for d in descs(c, buf):
            d.start()

    start(0, 0)
    start(1, 1)

    def pair_body(i, _):
        for sub in range(2):
            c = 2 * i + sub
            buf = sub
            for d in descs(c, buf):
                d.wait()

            def row_body(r, _):
                for j in range(H // LANES):
                    sl = pl.ds(j * LANES, LANES)
                    p = pos_v[buf, r, sl]
                    for b in range(B):
                        sb_v[buf, b, r, sl] = sb_v[buf, b, r, sl] + p
                return 0

            lax.fori_loop(0, P, row_body, 0)

            for b in range(B):
                pltpu.sync_copy(sb_v.at[buf, b],
                                out_hbm.at[b, pl.ds(s0 + c * P, P)])

            @pl.when(c + 2 < NCH)
            def _():
                start(c + 2, buf)
        return 0

    lax.fori_loop(0, NCH // 2, pair_body, 0)


@jax.jit
def _emb(x2d, table, pos):
    mesh = plsc.VectorSubcoreMesh(core_axis_name="c", subcore_axis_name="s")
    f = functools.partial(
        pl.kernel,
        mesh=mesh,
        out_type=jax.ShapeDtypeStruct((B, S, H), jnp.float32),
        scratch_types=[
            pltpu.VMEM((B * SEQ_PER_W,), jnp.int32),
            pltpu.VMEM((2, B, P, H), jnp.float32),
            pltpu.VMEM((2, P, H), jnp.float32),
            pltpu.SemaphoreType.DMA,
            pltpu.SemaphoreType.DMA,
        ],
    )(_emb_body)
    return f(x2d, table, pos)


def kernel(x, token_table, pos_emb):
    pos = pos_emb.reshape(S, H)
    return _emb(x.astype(jnp.int32), token_table, pos)


# speedup vs baseline: 1.7112x; 1.0166x over previous
import functools

import jax
import jax.numpy as jnp
from jax import lax
from jax.experimental import pallas as pl
from jax.experimental.pallas import tpu as pltpu
from jax.experimental.pallas import tpu_sc as plsc

B, S, H, V = 4, 2048, 1024, 50257
NC, NS = 2, 16
NW = NC * NS
SEQ_PER_W = S // NW
P = 8
NCH = SEQ_PER_W // P
LANES = 16
UNROLL = 8


def _emb_body(x_hbm, tab_hbm, pos_hbm, out_hbm, idx_v, sb_v, pos_v, sem0, sem1):
    wid = lax.axis_index("s") * NC + lax.axis_index("c")
    s0 = wid * SEQ_PER_W
    sems = (sem0, sem1)

    for b in range(B):
        pltpu.sync_copy(x_hbm.at[b, pl.ds(s0, SEQ_PER_W)],
                        idx_v.at[pl.ds(b * SEQ_PER_W, SEQ_PER_W)])

    def descs(c, buf):
        d = [pltpu.make_async_copy(pos_hbm.at[pl.ds(s0 + c * P, P)],
                                   pos_v.at[buf], sems[buf])]
        for b in range(B):
            d.append(pltpu.make_async_copy(
                tab_hbm.at[idx_v.at[pl.ds(b * SEQ_PER_W + c * P, P)]],
                sb_v.at[buf, b], sems[buf]))
        return d

    def start(c, buf):
        for d in descs(c, buf):
            d.start()

    start(0, 0)
    start(1, 1)

    def pair_body(i, _):
        for sub in range(2):
            c = 2 * i + sub
            buf = sub
            for d in descs(c, buf):
                d.wait()

            @plsc.parallel_loop(0, P * (H // LANES), unroll=UNROLL)
            def _(k):
                r = lax.shift_right_logical(k, 6)
                off = pl.multiple_of(
                    lax.shift_left(lax.bitwise_and(k, 63), 4), LANES)
                sl = pl.ds(off, LANES)
                p = pos_v[buf, r, sl]
                for b in range(B):
                    sb_v[buf, b, r, sl] = sb_v[buf, b, r, sl] + p

            for b in range(B):
                pltpu.sync_copy(sb_v.at[buf, b],
                                out_hbm.at[b, pl.ds(s0 + c * P, P)])

            @pl.when(c + 2 < NCH)
            def _():
                start(c + 2, buf)
        return 0

    lax.fori_loop(0, NCH // 2, pair_body, 0)


@jax.jit
def _emb(x2d, table, pos):
    mesh = plsc.VectorSubcoreMesh(core_axis_name="c", subcore_axis_name="s")
    f = functools.partial(
        pl.kernel,
        mesh=mesh,
        out_type=jax.ShapeDtypeStruct((B, S, H), jnp.float32),
        scratch_types=[
            pltpu.VMEM((B * SEQ_PER_W,), jnp.int32),
            pltpu.VMEM((2, B, P, H), jnp.float32),
            pltpu.VMEM((2, P, H), jnp.float32),
            pltpu.SemaphoreType.DMA,
            pltpu.SemaphoreType.DMA,
        ],
    )(_emb_body)
    return f(x2d, table, pos)


def kernel(x, token_table, pos_emb):
    pos = pos_emb.reshape(S, H)
    return _emb(x.astype(jnp.int32), token_table, pos)
